# Initial kernel scaffold; baseline (speedup 1.0000x reference)
import jax
import jax.numpy as jnp
from jax.experimental import pallas as pl


def kernel(ppi_x, metagraph_x, ppi_edgetypes, metagraph_edgetypes, ppi_edge_index, metagraph_edge_index, tissue_neighbors, ppi_W, ppi_att_src, ppi_att_dst, ppi_bias, meta_W, meta_att_src, meta_att_dst, meta_bias, sem_W, sem_b, sem_q, init_cci):
    raise NotImplementedError("write your pallas kernel here")


# speedup vs baseline: 35.7090x; 35.7090x over previous
import functools

import jax
import jax.numpy as jnp
from jax import lax
from jax.experimental import pallas as pl
from jax.experimental.pallas import tpu as pltpu
from jax.experimental.pallas import tpu_sc as plsc

C = 4; N = 10000; D = 128; R = 3; E = 160000
H = 3; OC = 64; HID = H * OC
RM = 2; T = 3; M = C + T; EM = 20; SEM = 128
TISSUE_UPDATE = 100

NP = 10048
TRASH = 10008
NC = 2; NS = 16; NW = NC * NS
RPT = NP // NW
KC = 64
EPT = 5056
NCH = EPT // KC
BN = 2512
NB = NP // BN
BNN = 1256
NBN = NP // BNN


def _prep_body(x_ref, w_ref, asf_ref, adf_ref, h_ref, as_ref, ad_ref):
    x = x_ref[0]
    w = w_ref[0]
    h = jnp.dot(x, w, preferred_element_type=jnp.float32)
    h_ref[0, 0] = h
    r = pl.program_id(1)
    asf = asf_ref[pl.ds(r, 1), :]
    adf = adf_ref[pl.ds(r, 1), :]
    ts = h * asf
    td = h * adf
    cols_s = []
    cols_d = []
    for hh in range(H):
        sl = slice(hh * OC, (hh + 1) * OC)
        cols_s.append(jnp.sum(ts[:, sl], axis=1, keepdims=True))
        cols_d.append(jnp.sum(td[:, sl], axis=1, keepdims=True))
    z = jnp.zeros((x.shape[0], 16 - H), jnp.float32)
    as_ref[0, 0] = jnp.concatenate(cols_s + [z], axis=1)
    ad_ref[0, 0] = jnp.concatenate(cols_d + [z], axis=1)


def _prep_call(xp, ppi_W, att_src_flat, att_dst_flat):
    return pl.pallas_call(
        _prep_body,
        grid=(C, R, NB),
        in_specs=[
            pl.BlockSpec((1, BN, D), lambda c, r, b: (c, b, 0)),
            pl.BlockSpec((1, D, HID), lambda c, r, b: (r, 0, 0)),
            pl.BlockSpec((R, HID), lambda c, r, b: (0, 0)),
            pl.BlockSpec((R, HID), lambda c, r, b: (0, 0)),
        ],
        out_specs=[
            pl.BlockSpec((1, 1, BN, HID), lambda c, r, b: (c, r, b, 0)),
            pl.BlockSpec((1, 1, BN, 16), lambda c, r, b: (c, r, b, 0)),
            pl.BlockSpec((1, 1, BN, 16), lambda c, r, b: (c, r, b, 0)),
        ],
        out_shape=[
            jax.ShapeDtypeStruct((C, R, NP, HID), jnp.float32),
            jax.ShapeDtypeStruct((C, R, NP, 16), jnp.float32),
            jax.ShapeDtypeStruct((C, R, NP, 16), jnp.float32),
        ],
    )(xp, ppi_W, att_src_flat, att_dst_flat)


def _edge_body(h3_hbm, asrc_hbm, adst_hbm, srch_hbm, srcg_hbm, dstg_hbm,
               dstl_hbm, numer_out, denom_out,
               idx_h, idx_g, idx_dg, idx_dl, rows_v, a1_v, a2_v, ex_v, zden_v,
               numer_sh, denom_sh, sem, sem2, sem3):
    cid = lax.axis_index("c")
    sid = lax.axis_index("s")
    tile = cid * NS + sid
    nblk = NP // KC
    f32 = jnp.float32

    def _zden(i, _):
        zden_v[i, :] = jnp.zeros((16,), f32)
        return 0
    lax.fori_loop(0, KC, _zden, 0)

    def cr_body(j, _):
        c = j // R
        r = j - c * R
        pltpu.sync_copy(srcg_hbm.at[c, r, tile], idx_g)
        pltpu.sync_copy(dstg_hbm.at[c, r, tile], idx_dg)
        pltpu.sync_copy(dstl_hbm.at[c, r, tile], idx_dl)

        for hp in range(H):
            def _zrows(i, _):
                rr = i // (OC // 16)
                qq = i - rr * (OC // 16)
                rows_v[rr, pl.ds(qq * 16, 16)] = jnp.zeros((16,), f32)
                return 0
            lax.fori_loop(0, KC * (OC // 16), _zrows, 0)

            def zblk(i, _):
                blk = sid + i * NS
                @pl.when(blk < nblk)
                def _():
                    pltpu.sync_copy(rows_v, numer_sh.at[pl.ds(blk * KC, KC)])
                    if hp == 0:
                        pltpu.sync_copy(zden_v,
                                        denom_sh.at[pl.ds(blk * KC, KC)])
                return 0
            lax.fori_loop(0, (nblk + NS - 1) // NS, zblk, 0)
            pltpu.sync_copy(srch_hbm.at[c, r, hp, tile], idx_h)
            plsc.subcore_barrier()

            def chunk(ci, _):
                dstv = idx_dl.at[ci, 0]
                dh = pltpu.async_copy(h3_hbm.at[idx_h.at[ci, 0]], rows_v, sem)
                da = pltpu.async_copy(asrc_hbm.at[idx_g.at[ci, 0]], a1_v, sem2)
                db = pltpu.async_copy(adst_hbm.at[idx_dg.at[ci, 0]], a2_v,
                                      sem3)
                da.wait()
                db.wait()
                dh.wait()
                for e in range(KC):
                    xx = a1_v[e, :] + a2_v[e, :]
                    exrow = jnp.exp(jnp.maximum(xx, 0.2 * xx))
                    if hp == 0:
                        ex_v[e, :] = exrow
                    cv = jnp.full((16,), exrow[hp])
                    for q in range(OC // 16):
                        rows_v[e, pl.ds(q * 16, 16)] = (
                            rows_v[e, pl.ds(q * 16, 16)] * cv)
                if hp == 0:
                    pltpu.sync_copy(ex_v, denom_sh.at[dstv], add=True)
                pltpu.sync_copy(rows_v, numer_sh.at[dstv], add=True)
                return 0
            lax.fori_loop(0, NCH, chunk, 0)
            plsc.subcore_barrier()

            def dblk(i, _):
                blk = sid + i * NS
                @pl.when(blk < nblk)
                def _():
                    pltpu.sync_copy(
                        numer_sh.at[pl.ds(blk * KC, KC)],
                        numer_out.at[c, r, hp, cid, pl.ds(blk * KC, KC)])
                    if hp == 0:
                        pltpu.sync_copy(
                            denom_sh.at[pl.ds(blk * KC, KC)],
                            denom_out.at[c, r, cid, pl.ds(blk * KC, KC)])
                return 0
            lax.fori_loop(0, (nblk + NS - 1) // NS, dblk, 0)
            plsc.subcore_barrier()
        return 0
    lax.fori_loop(0, C * R, cr_body, 0)


def _edge_kernel():
    mesh = plsc.VectorSubcoreMesh(core_axis_name="c", subcore_axis_name="s",
                                  num_cores=NC, num_subcores=NS)
    return pl.kernel(
        _edge_body,
        compiler_params=pltpu.CompilerParams(use_tc_tiling_on_sc=False),
        out_type=[
            jax.ShapeDtypeStruct((C, R, H, NC, NP, OC), jnp.float32),
            jax.ShapeDtypeStruct((C, R, NC, NP, 16), jnp.float32),
        ],
        mesh=mesh,
        scratch_types=[
            pltpu.VMEM((NCH, 1, KC), jnp.int32),
            pltpu.VMEM((NCH, 1, KC), jnp.int32),
            pltpu.VMEM((NCH, 1, KC), jnp.int32),
            pltpu.VMEM((NCH, 1, KC), jnp.int32),
            pltpu.VMEM((KC, OC), jnp.float32),
            pltpu.VMEM((KC, 16), jnp.float32),
            pltpu.VMEM((KC, 16), jnp.float32),
            pltpu.VMEM((KC, 16), jnp.float32),
            pltpu.VMEM((KC, 16), jnp.float32),
            pltpu.VMEM_SHARED((NP, OC), jnp.float32),
            pltpu.VMEM_SHARED((NP, 16), jnp.float32),
            pltpu.SemaphoreType.DMA,
            pltpu.SemaphoreType.DMA,
            pltpu.SemaphoreType.DMA,
        ],
    )


def _edge_call(h_all, asrc, adst, srch, srcg, dstg, dstl):
    h3 = h_all.reshape(C * R * NP * H, OC)
    return _edge_kernel()(h3, asrc.reshape(C * R * NP, 16),
                          adst.reshape(C * R * NP, 16),
                          srch, srcg, dstg, dstl)


def _norm_body(num_ref, den_ref, h_ref, as_ref, ad_ref, bias_ref,
               sw_ref, sb_ref, sq_ref, z_ref, m_ref):
    b = pl.program_id(1)
    Os = []
    for r in range(R):
        dn = den_ref[0, r, 0] + den_ref[0, r, 1]
        asl = as_ref[0, r]
        adl = ad_ref[0, r]
        xx = asl + adl
        exs = jnp.exp(jnp.maximum(xx, 0.2 * xx))
        hrow = h_ref[0, r]
        cols = []
        for hh in range(H):
            sl = slice(hh * OC, (hh + 1) * OC)
            e_h = exs[:, hh:hh + 1]
            nm = num_ref[0, r, hh, 0] + num_ref[0, r, hh, 1]
            numer = nm + hrow[:, sl] * e_h
            denom = dn[:, hh:hh + 1] + e_h
            cols.append(numer / (denom + 1e-16))
        out = jnp.concatenate(cols, axis=1) + bias_ref[r:r + 1, :]
        Os.append(jax.nn.relu(out))
    sw = sw_ref[...]
    sb = sb_ref[0:1, :]
    sq = sq_ref[0:1, :]
    betas = []
    for r in range(R):
        w = jnp.tanh(jnp.dot(Os[r], sw, preferred_element_type=jnp.float32) + sb)
        betas.append(jnp.sum(w * sq, axis=1, keepdims=True))
    bmax = jnp.maximum(jnp.maximum(betas[0], betas[1]), betas[2])
    es = [jnp.exp(bb - bmax) for bb in betas]
    esum = es[0] + es[1] + es[2]
    z = (Os[0] * es[0] + Os[1] * es[1] + Os[2] * es[2]) / esum
    z_ref[0] = z
    rows = jax.lax.broadcasted_iota(jnp.int32, (BNN, 1), 0) + b * BNN
    zm = jnp.where(rows < N, z, 0.0)
    contrib = jnp.sum(zm, axis=0, keepdims=True) * (1.0 / N)
    @pl.when(b == 0)
    def _():
        m_ref[0] = contrib
    @pl.when(b != 0)
    def _():
        m_ref[0] = m_ref[0] + contrib


def _norm_call(numer, denom, h_all, asrc, adst, ppi_bias, sw, sb, sq):
    return pl.pallas_call(
        _norm_body,
        grid=(C, NBN),
        in_specs=[
            pl.BlockSpec((1, R, H, NC, BNN, OC), lambda c, b: (c, 0, 0, 0, b, 0)),
            pl.BlockSpec((1, R, NC, BNN, 16), lambda c, b: (c, 0, 0, b, 0)),
            pl.BlockSpec((1, R, BNN, HID), lambda c, b: (c, 0, b, 0)),
            pl.BlockSpec((1, R, BNN, 16), lambda c, b: (c, 0, b, 0)),
            pl.BlockSpec((1, R, BNN, 16), lambda c, b: (c, 0, b, 0)),
            pl.BlockSpec((R, HID), lambda c, b: (0, 0)),
            pl.BlockSpec((HID, SEM), lambda c, b: (0, 0)),
            pl.BlockSpec((1, SEM), lambda c, b: (0, 0)),
            pl.BlockSpec((1, SEM), lambda c, b: (0, 0)),
        ],
        out_specs=[
            pl.BlockSpec((1, BNN, HID), lambda c, b: (c, b, 0)),
            pl.BlockSpec((1, 1, HID), lambda c, b: (c, 0, 0)),
        ],
        out_shape=[
            jax.ShapeDtypeStruct((C, NP, HID), jnp.float32),
            jax.ShapeDtypeStruct((C, 1, HID), jnp.float32),
        ],
    )(numer, denom, h_all, asrc, adst, ppi_bias, sw, sb, sq)


def _meta_body(m_ref, nbr_ref, cnt_ref, mw_ref, mas_ref, mad_ref, mb_ref,
               sw_ref, sb_ref, sq_ref, out_ref):
    MP = 8
    meta0 = jnp.concatenate(
        [m_ref[...], jnp.zeros((MP - C, HID), jnp.float32)], axis=0)

    def smooth(_, meta):
        for ti in range(T):
            newrow = jnp.dot(nbr_ref[ti:ti + 1, :], meta,
                             preferred_element_type=jnp.float32)
            rmask = jax.lax.broadcasted_iota(jnp.int32, (MP, HID), 0) == (C + ti)
            meta = jnp.where(rmask, newrow, meta)
        return meta
    meta = lax.fori_loop(0, TISSUE_UPDATE, smooth, meta0)

    ones_col = jnp.ones((MP, 1), jnp.float32)
    Os = []
    for r in range(RM):
        hm = jnp.dot(meta, mw_ref[r], preferred_element_type=jnp.float32)
        tsr = hm * mas_ref[r:r + 1, :]
        tdr = hm * mad_ref[r:r + 1, :]
        cnt = cnt_ref[r]
        cols = []
        for hh in range(H):
            sl = slice(hh * OC, (hh + 1) * OC)
            asr = jnp.sum(tsr[:, sl], axis=1, keepdims=True)
            adr = jnp.sum(tdr[:, sl], axis=1, keepdims=True)
            asr_mat = lax.dot_general(ones_col, asr,
                                      (((1,), (1,)), ((), ())),
                                      preferred_element_type=jnp.float32)
            xx = asr_mat + adr
            wgt = cnt * jnp.exp(jnp.maximum(xx, 0.2 * xx))
            den = jnp.sum(wgt, axis=1, keepdims=True)
            num = jnp.dot(wgt, hm[:, sl], preferred_element_type=jnp.float32)
            cols.append(num / (den + 1e-16))
        out = jnp.concatenate(cols, axis=1) + mb_ref[r:r + 1, :]
        Os.append(jax.nn.relu(out))
    sw = sw_ref[...]
    sb = sb_ref[0:1, :]
    sq = sq_ref[0:1, :]
    betas = []
    for r in range(RM):
        w = jnp.tanh(jnp.dot(Os[r], sw, preferred_element_type=jnp.float32) + sb)
        betas.append(jnp.sum(w * sq, axis=1, keepdims=True))
    bmax = jnp.maximum(betas[0], betas[1])
    es = [jnp.exp(bb - bmax) for bb in betas]
    esum = es[0] + es[1]
    out_ref[...] = (Os[0] * es[0] + Os[1] * es[1]) / esum


def _meta_call(m, nbr_oh, cnt, meta_W, mas_flat, mad_flat, meta_bias, sw, sb, sq):
    return pl.pallas_call(
        _meta_body,
        out_shape=jax.ShapeDtypeStruct((8, HID), jnp.float32),
    )(m, nbr_oh, cnt, meta_W, mas_flat, mad_flat, meta_bias, sw, sb, sq)


def kernel(ppi_x, metagraph_x, ppi_edgetypes, metagraph_edgetypes,
           ppi_edge_index, metagraph_edge_index, tissue_neighbors,
           ppi_W, ppi_att_src, ppi_att_dst, ppi_bias,
           meta_W, meta_att_src, meta_att_dst, meta_bias,
           sem_W, sem_b, sem_q, init_cci):
    f32 = jnp.float32
    ppi_x = ppi_x.astype(f32)
    xp = jnp.pad(ppi_x, ((0, 0), (0, NP - N), (0, 0)))

    et = ppi_edgetypes.astype(jnp.int32)
    pad_e = NW * EPT - E
    src = jnp.pad(et[:, :, 0, :], ((0, 0), (0, 0), (0, pad_e)),
                  constant_values=TRASH)
    dst = jnp.pad(et[:, :, 1, :], ((0, 0), (0, 0), (0, pad_e)),
                  constant_values=TRASH)
    offs = (jnp.arange(C)[:, None] * R + jnp.arange(R)[None, :]) * NP
    gsrc = src + offs[:, :, None]
    gdst = dst + offs[:, :, None]
    srcg = gsrc.reshape(C, R, NW, NCH, 1, KC)
    dstg = gdst.reshape(C, R, NW, NCH, 1, KC)
    dstl = dst.reshape(C, R, NW, NCH, 1, KC)
    srch = (gsrc[:, :, None, :] * H
            + jnp.arange(H)[None, None, :, None]).reshape(
                C, R, H, NW, NCH, 1, KC)

    att_src_flat = ppi_att_src.reshape(R, HID).astype(f32)
    att_dst_flat = ppi_att_dst.reshape(R, HID).astype(f32)
    sw = sem_W.reshape(HID, SEM).astype(f32)
    sb = sem_b.reshape(1, SEM).astype(f32)
    sq = sem_q.reshape(1, SEM).astype(f32)

    h_all, asrc, adst = _prep_call(xp, ppi_W.astype(f32),
                                   att_src_flat, att_dst_flat)
    numer, denom = _edge_call(h_all, asrc, adst, srch, srcg, dstg, dstl)
    z, m = _norm_call(numer, denom, h_all, asrc, adst,
                      ppi_bias.astype(f32), sw, sb, sq)
    m = m.reshape(C, HID)

    rows8 = jnp.arange(8)
    nbr_oh = (jnp.sum((rows8[None, None, :] == tissue_neighbors[:, :, None])
                      .astype(f32), axis=1) / 3.0)
    me = metagraph_edgetypes.astype(jnp.int32)
    cnt = jnp.sum((rows8[None, :, None, None] == me[:, None, None, 1, :])
                  & (rows8[None, None, :, None] == me[:, None, None, 0, :]),
                  axis=-1).astype(f32)
    cnt = cnt + jnp.eye(8, dtype=f32)[None]

    meta_out = _meta_call(m, nbr_oh, cnt, meta_W.astype(f32),
                          meta_att_src.reshape(RM, HID).astype(f32),
                          meta_att_dst.reshape(RM, HID).astype(f32),
                          meta_bias.astype(f32), sw, sb, sq)

    return z[:, :N, :], meta_out[:M]


# speedup vs baseline: 41.6118x; 1.1653x over previous
import functools

import jax
import jax.numpy as jnp
from jax import lax
from jax.experimental import pallas as pl
from jax.experimental.pallas import tpu as pltpu
from jax.experimental.pallas import tpu_sc as plsc

C = 4; N = 10000; D = 128; R = 3; E = 160000
H = 3; OC = 64; HID = H * OC
RM = 2; T = 3; M = C + T; EM = 20; SEM = 128
TISSUE_UPDATE = 100

NP = 10048
TRASH = 10008
NC = 2; NS = 16; NW = NC * NS
RPT = NP // NW
KC = 64
EPT = 5120
NCH = EPT // KC
BN = 2512
NB = NP // BN
BNN = 1256
NBN = NP // BNN


def _prep_body(x_ref, w_ref, asf_ref, adf_ref, h_ref, as_ref, ad_ref):
    x = x_ref[0]
    w = w_ref[0]
    h = jnp.dot(x, w, preferred_element_type=jnp.float32)
    h_ref[0, 0] = h
    r = pl.program_id(1)
    asf = asf_ref[pl.ds(r, 1), :]
    adf = adf_ref[pl.ds(r, 1), :]
    ts = h * asf
    td = h * adf
    cols_s = []
    cols_d = []
    for hh in range(H):
        sl = slice(hh * OC, (hh + 1) * OC)
        cols_s.append(jnp.sum(ts[:, sl], axis=1, keepdims=True))
        cols_d.append(jnp.sum(td[:, sl], axis=1, keepdims=True))
    z = jnp.zeros((x.shape[0], 16 - H), jnp.float32)
    as_ref[0, 0] = jnp.concatenate(cols_s + [z], axis=1)
    ad_ref[0, 0] = jnp.concatenate(cols_d + [z], axis=1)


def _prep_call(xp, ppi_W, att_src_flat, att_dst_flat):
    return pl.pallas_call(
        _prep_body,
        grid=(C, R, NB),
        in_specs=[
            pl.BlockSpec((1, BN, D), lambda c, r, b: (c, b, 0)),
            pl.BlockSpec((1, D, HID), lambda c, r, b: (r, 0, 0)),
            pl.BlockSpec((R, HID), lambda c, r, b: (0, 0)),
            pl.BlockSpec((R, HID), lambda c, r, b: (0, 0)),
        ],
        out_specs=[
            pl.BlockSpec((1, 1, BN, HID), lambda c, r, b: (c, r, b, 0)),
            pl.BlockSpec((1, 1, BN, 16), lambda c, r, b: (c, r, b, 0)),
            pl.BlockSpec((1, 1, BN, 16), lambda c, r, b: (c, r, b, 0)),
        ],
        out_shape=[
            jax.ShapeDtypeStruct((C, R, NP, HID), jnp.float32),
            jax.ShapeDtypeStruct((C, R, NP, 16), jnp.float32),
            jax.ShapeDtypeStruct((C, R, NP, 16), jnp.float32),
        ],
    )(xp, ppi_W, att_src_flat, att_dst_flat)


def _edge_body(h3_hbm, asrc_hbm, adst_hbm, srch_hbm, srcg_hbm, dstg_hbm,
               dstl_hbm, numer_out, denom_out,
               idx_h, idx_g, idx_dg, idx_dl, rows_v0, rows_v1, a1_v0, a1_v1,
               a2_v0, a2_v1, ex_v, zden_v, numer_sh, denom_sh, sem0, sem1):
    rows_b = [rows_v0, rows_v1]
    a1_b = [a1_v0, a1_v1]
    a2_b = [a2_v0, a2_v1]
    sem_b = [sem0, sem1]
    cid = lax.axis_index("c")
    sid = lax.axis_index("s")
    tile = cid * NS + sid
    nblk = NP // KC
    f32 = jnp.float32

    def _zden(i, _):
        zden_v[i, :] = jnp.zeros((16,), f32)
        return 0
    lax.fori_loop(0, KC, _zden, 0)

    def cr_body(j, _):
        c = j // R
        r = j - c * R
        pltpu.sync_copy(srcg_hbm.at[c, r, tile], idx_g)
        pltpu.sync_copy(dstg_hbm.at[c, r, tile], idx_dg)
        pltpu.sync_copy(dstl_hbm.at[c, r, tile], idx_dl)

        for hp in range(H):
            def _zrows(i, _):
                rr = i // (OC // 16)
                qq = i - rr * (OC // 16)
                rows_v0[rr, pl.ds(qq * 16, 16)] = jnp.zeros((16,), f32)
                return 0
            lax.fori_loop(0, KC * (OC // 16), _zrows, 0)

            def zblk(i, _):
                blk = sid + i * NS
                @pl.when(blk < nblk)
                def _():
                    pltpu.sync_copy(rows_v0, numer_sh.at[pl.ds(blk * KC, KC)])
                    if hp == 0:
                        pltpu.sync_copy(zden_v,
                                        denom_sh.at[pl.ds(blk * KC, KC)])
                return 0
            lax.fori_loop(0, (nblk + NS - 1) // NS, zblk, 0)
            pltpu.sync_copy(srch_hbm.at[c, r, hp, tile], idx_h)
            plsc.subcore_barrier()

            def _start(ci, b):
                pltpu.async_copy(h3_hbm.at[idx_h.at[ci, 0]], rows_b[b],
                                 sem_b[b])
                pltpu.async_copy(asrc_hbm.at[idx_g.at[ci, 0]], a1_b[b],
                                 sem_b[b])
                pltpu.async_copy(adst_hbm.at[idx_dg.at[ci, 0]], a2_b[b],
                                 sem_b[b])

            def _wait(ci, b):
                pltpu.make_async_copy(h3_hbm.at[idx_h.at[ci, 0]], rows_b[b],
                                      sem_b[b]).wait()
                pltpu.make_async_copy(asrc_hbm.at[idx_g.at[ci, 0]], a1_b[b],
                                      sem_b[b]).wait()
                pltpu.make_async_copy(adst_hbm.at[idx_dg.at[ci, 0]], a2_b[b],
                                      sem_b[b]).wait()

            _start(0, 0)
            def chunk2(cio, _):
                for b in range(2):
                    ci = cio * 2 + b
                    nci = ci + 1
                    @pl.when(nci < NCH)
                    def _():
                        _start(nci, 1 - b)
                    _wait(ci, b)
                    rows_v = rows_b[b]
                    a1_v = a1_b[b]
                    a2_v = a2_b[b]
                    for e in range(KC):
                        xx = a1_v[e, :] + a2_v[e, :]
                        exrow = jnp.exp(jnp.maximum(xx, 0.2 * xx))
                        if hp == 0:
                            ex_v[e, :] = exrow
                        cv = jnp.full((16,), exrow[hp])
                        for q in range(OC // 16):
                            rows_v[e, pl.ds(q * 16, 16)] = (
                                rows_v[e, pl.ds(q * 16, 16)] * cv)
                    dstv = idx_dl.at[ci, 0]
                    if hp == 0:
                        pltpu.sync_copy(ex_v, denom_sh.at[dstv], add=True)
                    pltpu.sync_copy(rows_v, numer_sh.at[dstv], add=True)
                return 0
            lax.fori_loop(0, NCH // 2, chunk2, 0)
            plsc.subcore_barrier()

            def dblk(i, _):
                blk = sid + i * NS
                @pl.when(blk < nblk)
                def _():
                    pltpu.sync_copy(
                        numer_sh.at[pl.ds(blk * KC, KC)],
                        numer_out.at[c, r, hp, cid, pl.ds(blk * KC, KC)])
                    if hp == 0:
                        pltpu.sync_copy(
                            denom_sh.at[pl.ds(blk * KC, KC)],
                            denom_out.at[c, r, cid, pl.ds(blk * KC, KC)])
                return 0
            lax.fori_loop(0, (nblk + NS - 1) // NS, dblk, 0)
            plsc.subcore_barrier()
        return 0
    lax.fori_loop(0, C * R, cr_body, 0)


def _edge_kernel():
    mesh = plsc.VectorSubcoreMesh(core_axis_name="c", subcore_axis_name="s",
                                  num_cores=NC, num_subcores=NS)
    return pl.kernel(
        _edge_body,
        compiler_params=pltpu.CompilerParams(use_tc_tiling_on_sc=False),
        out_type=[
            jax.ShapeDtypeStruct((C, R, H, NC, NP, OC), jnp.float32),
            jax.ShapeDtypeStruct((C, R, NC, NP, 16), jnp.float32),
        ],
        mesh=mesh,
        scratch_types=[
            pltpu.VMEM((NCH, 1, KC), jnp.int32),
            pltpu.VMEM((NCH, 1, KC), jnp.int32),
            pltpu.VMEM((NCH, 1, KC), jnp.int32),
            pltpu.VMEM((NCH, 1, KC), jnp.int32),
            pltpu.VMEM((KC, OC), jnp.float32),
            pltpu.VMEM((KC, OC), jnp.float32),
            pltpu.VMEM((KC, 16), jnp.float32),
            pltpu.VMEM((KC, 16), jnp.float32),
            pltpu.VMEM((KC, 16), jnp.float32),
            pltpu.VMEM((KC, 16), jnp.float32),
            pltpu.VMEM((KC, 16), jnp.float32),
            pltpu.VMEM((KC, 16), jnp.float32),
            pltpu.VMEM_SHARED((NP, OC), jnp.float32),
            pltpu.VMEM_SHARED((NP, 16), jnp.float32),
            pltpu.SemaphoreType.DMA,
            pltpu.SemaphoreType.DMA,
        ],
    )


def _edge_call(h_all, asrc, adst, srch, srcg, dstg, dstl):
    h3 = h_all.reshape(C * R * NP * H, OC)
    return _edge_kernel()(h3, asrc.reshape(C * R * NP, 16),
                          adst.reshape(C * R * NP, 16),
                          srch, srcg, dstg, dstl)


def _norm_body(num_ref, den_ref, h_ref, as_ref, ad_ref, bias_ref,
               sw_ref, sb_ref, sq_ref, z_ref, m_ref):
    b = pl.program_id(1)
    Os = []
    for r in range(R):
        dn = den_ref[0, r, 0] + den_ref[0, r, 1]
        asl = as_ref[0, r]
        adl = ad_ref[0, r]
        xx = asl + adl
        exs = jnp.exp(jnp.maximum(xx, 0.2 * xx))
        hrow = h_ref[0, r]
        cols = []
        for hh in range(H):
            sl = slice(hh * OC, (hh + 1) * OC)
            e_h = exs[:, hh:hh + 1]
            nm = num_ref[0, r, hh, 0] + num_ref[0, r, hh, 1]
            numer = nm + hrow[:, sl] * e_h
            denom = dn[:, hh:hh + 1] + e_h
            cols.append(numer / (denom + 1e-16))
        out = jnp.concatenate(cols, axis=1) + bias_ref[r:r + 1, :]
        Os.append(jax.nn.relu(out))
    sw = sw_ref[...]
    sb = sb_ref[0:1, :]
    sq = sq_ref[0:1, :]
    betas = []
    for r in range(R):
        w = jnp.tanh(jnp.dot(Os[r], sw, preferred_element_type=jnp.float32) + sb)
        betas.append(jnp.sum(w * sq, axis=1, keepdims=True))
    bmax = jnp.maximum(jnp.maximum(betas[0], betas[1]), betas[2])
    es = [jnp.exp(bb - bmax) for bb in betas]
    esum = es[0] + es[1] + es[2]
    z = (Os[0] * es[0] + Os[1] * es[1] + Os[2] * es[2]) / esum
    z_ref[0] = z
    rows = jax.lax.broadcasted_iota(jnp.int32, (BNN, 1), 0) + b * BNN
    zm = jnp.where(rows < N, z, 0.0)
    contrib = jnp.sum(zm, axis=0, keepdims=True) * (1.0 / N)
    @pl.when(b == 0)
    def _():
        m_ref[0] = contrib
    @pl.when(b != 0)
    def _():
        m_ref[0] = m_ref[0] + contrib


def _norm_call(numer, denom, h_all, asrc, adst, ppi_bias, sw, sb, sq):
    return pl.pallas_call(
        _norm_body,
        grid=(C, NBN),
        in_specs=[
            pl.BlockSpec((1, R, H, NC, BNN, OC), lambda c, b: (c, 0, 0, 0, b, 0)),
            pl.BlockSpec((1, R, NC, BNN, 16), lambda c, b: (c, 0, 0, b, 0)),
            pl.BlockSpec((1, R, BNN, HID), lambda c, b: (c, 0, b, 0)),
            pl.BlockSpec((1, R, BNN, 16), lambda c, b: (c, 0, b, 0)),
            pl.BlockSpec((1, R, BNN, 16), lambda c, b: (c, 0, b, 0)),
            pl.BlockSpec((R, HID), lambda c, b: (0, 0)),
            pl.BlockSpec((HID, SEM), lambda c, b: (0, 0)),
            pl.BlockSpec((1, SEM), lambda c, b: (0, 0)),
            pl.BlockSpec((1, SEM), lambda c, b: (0, 0)),
        ],
        out_specs=[
            pl.BlockSpec((1, BNN, HID), lambda c, b: (c, b, 0)),
            pl.BlockSpec((1, 1, HID), lambda c, b: (c, 0, 0)),
        ],
        out_shape=[
            jax.ShapeDtypeStruct((C, NP, HID), jnp.float32),
            jax.ShapeDtypeStruct((C, 1, HID), jnp.float32),
        ],
    )(numer, denom, h_all, asrc, adst, ppi_bias, sw, sb, sq)


def _meta_body(m_ref, nbr_ref, cnt_ref, mw_ref, mas_ref, mad_ref, mb_ref,
               sw_ref, sb_ref, sq_ref, out_ref):
    MP = 8
    meta0 = jnp.concatenate(
        [m_ref[...], jnp.zeros((MP - C, HID), jnp.float32)], axis=0)

    def smooth(_, meta):
        for ti in range(T):
            newrow = jnp.dot(nbr_ref[ti:ti + 1, :], meta,
                             preferred_element_type=jnp.float32)
            rmask = jax.lax.broadcasted_iota(jnp.int32, (MP, HID), 0) == (C + ti)
            meta = jnp.where(rmask, newrow, meta)
        return meta
    meta = lax.fori_loop(0, TISSUE_UPDATE, smooth, meta0)

    ones_col = jnp.ones((MP, 1), jnp.float32)
    Os = []
    for r in range(RM):
        hm = jnp.dot(meta, mw_ref[r], preferred_element_type=jnp.float32)
        tsr = hm * mas_ref[r:r + 1, :]
        tdr = hm * mad_ref[r:r + 1, :]
        cnt = cnt_ref[r]
        cols = []
        for hh in range(H):
            sl = slice(hh * OC, (hh + 1) * OC)
            asr = jnp.sum(tsr[:, sl], axis=1, keepdims=True)
            adr = jnp.sum(tdr[:, sl], axis=1, keepdims=True)
            asr_mat = lax.dot_general(ones_col, asr,
                                      (((1,), (1,)), ((), ())),
                                      preferred_element_type=jnp.float32)
            xx = asr_mat + adr
            wgt = cnt * jnp.exp(jnp.maximum(xx, 0.2 * xx))
            den = jnp.sum(wgt, axis=1, keepdims=True)
            num = jnp.dot(wgt, hm[:, sl], preferred_element_type=jnp.float32)
            cols.append(num / (den + 1e-16))
        out = jnp.concatenate(cols, axis=1) + mb_ref[r:r + 1, :]
        Os.append(jax.nn.relu(out))
    sw = sw_ref[...]
    sb = sb_ref[0:1, :]
    sq = sq_ref[0:1, :]
    betas = []
    for r in range(RM):
        w = jnp.tanh(jnp.dot(Os[r], sw, preferred_element_type=jnp.float32) + sb)
        betas.append(jnp.sum(w * sq, axis=1, keepdims=True))
    bmax = jnp.maximum(betas[0], betas[1])
    es = [jnp.exp(bb - bmax) for bb in betas]
    esum = es[0] + es[1]
    out_ref[...] = (Os[0] * es[0] + Os[1] * es[1]) / esum


def _meta_call(m, nbr_oh, cnt, meta_W, mas_flat, mad_flat, meta_bias, sw, sb, sq):
    return pl.pallas_call(
        _meta_body,
        out_shape=jax.ShapeDtypeStruct((8, HID), jnp.float32),
    )(m, nbr_oh, cnt, meta_W, mas_flat, mad_flat, meta_bias, sw, sb, sq)


def kernel(ppi_x, metagraph_x, ppi_edgetypes, metagraph_edgetypes,
           ppi_edge_index, metagraph_edge_index, tissue_neighbors,
           ppi_W, ppi_att_src, ppi_att_dst, ppi_bias,
           meta_W, meta_att_src, meta_att_dst, meta_bias,
           sem_W, sem_b, sem_q, init_cci):
    f32 = jnp.float32
    ppi_x = ppi_x.astype(f32)
    xp = jnp.pad(ppi_x, ((0, 0), (0, NP - N), (0, 0)))

    et = ppi_edgetypes.astype(jnp.int32)
    pad_e = NW * EPT - E
    src = jnp.pad(et[:, :, 0, :], ((0, 0), (0, 0), (0, pad_e)),
                  constant_values=TRASH)
    dst = jnp.pad(et[:, :, 1, :], ((0, 0), (0, 0), (0, pad_e)),
                  constant_values=TRASH)
    offs = (jnp.arange(C)[:, None] * R + jnp.arange(R)[None, :]) * NP
    gsrc = src + offs[:, :, None]
    gdst = dst + offs[:, :, None]
    srcg = gsrc.reshape(C, R, NW, NCH, 1, KC)
    dstg = gdst.reshape(C, R, NW, NCH, 1, KC)
    dstl = dst.reshape(C, R, NW, NCH, 1, KC)
    srch = (gsrc[:, :, None, :] * H
            + jnp.arange(H)[None, None, :, None]).reshape(
                C, R, H, NW, NCH, 1, KC)

    att_src_flat = ppi_att_src.reshape(R, HID).astype(f32)
    att_dst_flat = ppi_att_dst.reshape(R, HID).astype(f32)
    sw = sem_W.reshape(HID, SEM).astype(f32)
    sb = sem_b.reshape(1, SEM).astype(f32)
    sq = sem_q.reshape(1, SEM).astype(f32)

    h_all, asrc, adst = _prep_call(xp, ppi_W.astype(f32),
                                   att_src_flat, att_dst_flat)
    numer, denom = _edge_call(h_all, asrc, adst, srch, srcg, dstg, dstl)
    z, m = _norm_call(numer, denom, h_all, asrc, adst,
                      ppi_bias.astype(f32), sw, sb, sq)
    m = m.reshape(C, HID)

    rows8 = jnp.arange(8)
    nbr_oh = (jnp.sum((rows8[None, None, :] == tissue_neighbors[:, :, None])
                      .astype(f32), axis=1) / 3.0)
    me = metagraph_edgetypes.astype(jnp.int32)
    cnt = jnp.sum((rows8[None, :, None, None] == me[:, None, None, 1, :])
                  & (rows8[None, None, :, None] == me[:, None, None, 0, :]),
                  axis=-1).astype(f32)
    cnt = cnt + jnp.eye(8, dtype=f32)[None]

    meta_out = _meta_call(m, nbr_oh, cnt, meta_W.astype(f32),
                          meta_att_src.reshape(RM, HID).astype(f32),
                          meta_att_dst.reshape(RM, HID).astype(f32),
                          meta_bias.astype(f32), sw, sb, sq)

    return z[:, :N, :], meta_out[:M]


# speedup vs baseline: 41.6502x; 1.0009x over previous
import functools

import jax
import jax.numpy as jnp
from jax import lax
from jax.experimental import pallas as pl
from jax.experimental.pallas import tpu as pltpu
from jax.experimental.pallas import tpu_sc as plsc

C = 4; N = 10000; D = 128; R = 3; E = 160000
H = 3; OC = 64; HID = H * OC
RM = 2; T = 3; M = C + T; EM = 20; SEM = 128
TISSUE_UPDATE = 100

NP = 10048
TRASH = 10008
NC = 2; NS = 16; NW = NC * NS
RPT = NP // NW
KC = 64
EPT = 5120
NCH = EPT // KC
BN = 2512
NB = NP // BN
BNN = 1256
NBN = NP // BNN


def _prep_body(x_ref, w_ref, asf_ref, adf_ref, h_ref, as_ref, ad_ref):
    x = x_ref[0]
    w = w_ref[0]
    h = jnp.dot(x, w, preferred_element_type=jnp.float32)
    h_ref[0, 0] = h
    r = pl.program_id(1)
    asf = asf_ref[pl.ds(r, 1), :]
    adf = adf_ref[pl.ds(r, 1), :]
    ts = h * asf
    td = h * adf
    cols_s = []
    cols_d = []
    for hh in range(H):
        sl = slice(hh * OC, (hh + 1) * OC)
        cols_s.append(jnp.sum(ts[:, sl], axis=1, keepdims=True))
        cols_d.append(jnp.sum(td[:, sl], axis=1, keepdims=True))
    z = jnp.zeros((x.shape[0], 16 - H), jnp.float32)
    as_ref[0, 0] = jnp.concatenate(cols_s + [z], axis=1)
    ad_ref[0, 0] = jnp.concatenate(cols_d + [z], axis=1)


def _prep_call(xp, ppi_W, att_src_flat, att_dst_flat):
    return pl.pallas_call(
        _prep_body,
        grid=(C, R, NB),
        in_specs=[
            pl.BlockSpec((1, BN, D), lambda c, r, b: (c, b, 0)),
            pl.BlockSpec((1, D, HID), lambda c, r, b: (r, 0, 0)),
            pl.BlockSpec((R, HID), lambda c, r, b: (0, 0)),
            pl.BlockSpec((R, HID), lambda c, r, b: (0, 0)),
        ],
        out_specs=[
            pl.BlockSpec((1, 1, BN, HID), lambda c, r, b: (c, r, b, 0)),
            pl.BlockSpec((1, 1, BN, 16), lambda c, r, b: (c, r, b, 0)),
            pl.BlockSpec((1, 1, BN, 16), lambda c, r, b: (c, r, b, 0)),
        ],
        out_shape=[
            jax.ShapeDtypeStruct((C, R, NP, HID), jnp.float32),
            jax.ShapeDtypeStruct((C, R, NP, 16), jnp.float32),
            jax.ShapeDtypeStruct((C, R, NP, 16), jnp.float32),
        ],
    )(xp, ppi_W, att_src_flat, att_dst_flat)


def _edge_body(h3_hbm, asrc_hbm, adst_hbm, srch_hbm, srcg_hbm, dstg_hbm,
               dstl_hbm, numer_out, denom_out,
               idx_h, idx_g, idx_dg, idx_dl, rows_v0, rows_v1, a1_v0, a1_v1,
               a2_v0, a2_v1, ex_v0, ex_v1, zden_v, numer_sh, denom_sh,
               sem0, sem1, sem2, sem3):
    rows_b = [rows_v0, rows_v1]
    a1_b = [a1_v0, a1_v1]
    a2_b = [a2_v0, a2_v1]
    ex_b = [ex_v0, ex_v1]
    sem_b = [sem0, sem1]
    sems = [sem2, sem3]
    cid = lax.axis_index("c")
    sid = lax.axis_index("s")
    tile = cid * NS + sid
    nblk = NP // KC
    f32 = jnp.float32

    def _zden(i, _):
        zden_v[i, :] = jnp.zeros((16,), f32)
        return 0
    lax.fori_loop(0, KC, _zden, 0)

    def cr_body(j, _):
        c = j // R
        r = j - c * R
        pltpu.sync_copy(srcg_hbm.at[c, r, tile], idx_g)
        pltpu.sync_copy(dstg_hbm.at[c, r, tile], idx_dg)
        pltpu.sync_copy(dstl_hbm.at[c, r, tile], idx_dl)

        for hp in range(H):
            def _zrows(i, _):
                rr = i // (OC // 16)
                qq = i - rr * (OC // 16)
                rows_v0[rr, pl.ds(qq * 16, 16)] = jnp.zeros((16,), f32)
                return 0
            lax.fori_loop(0, KC * (OC // 16), _zrows, 0)

            def zblk(i, _):
                blk = sid + i * NS
                @pl.when(blk < nblk)
                def _():
                    pltpu.sync_copy(rows_v0, numer_sh.at[pl.ds(blk * KC, KC)])
                    if hp == 0:
                        pltpu.sync_copy(zden_v,
                                        denom_sh.at[pl.ds(blk * KC, KC)])
                return 0
            lax.fori_loop(0, (nblk + NS - 1) // NS, zblk, 0)
            pltpu.sync_copy(srch_hbm.at[c, r, hp, tile], idx_h)
            plsc.subcore_barrier()

            def _start(ci, b):
                pltpu.async_copy(h3_hbm.at[idx_h.at[ci, 0]], rows_b[b],
                                 sem_b[b])
                pltpu.async_copy(asrc_hbm.at[idx_g.at[ci, 0]], a1_b[b],
                                 sem_b[b])
                pltpu.async_copy(adst_hbm.at[idx_dg.at[ci, 0]], a2_b[b],
                                 sem_b[b])

            def _wait(ci, b):
                pltpu.make_async_copy(h3_hbm.at[idx_h.at[ci, 0]], rows_b[b],
                                      sem_b[b]).wait()
                pltpu.make_async_copy(asrc_hbm.at[idx_g.at[ci, 0]],
                                      a1_b[b], sem_b[b]).wait()
                pltpu.make_async_copy(adst_hbm.at[idx_dg.at[ci, 0]],
                                      a2_b[b], sem_b[b]).wait()

            def _scat_wait(x):
                pltpu.make_async_copy(rows_b[x],
                                      numer_sh.at[idx_dl.at[0, 0]],
                                      sems[x]).wait()
                if hp == 0:
                    pltpu.make_async_copy(ex_b[x],
                                          denom_sh.at[idx_dl.at[0, 0]],
                                          sems[x]).wait()

            _start(0, 0)
            def chunk2(cio, _):
                for b in range(2):
                    ci = cio * 2 + b
                    nci = ci + 1
                    @pl.when(ci > 0)
                    def _():
                        _scat_wait(1 - b)
                    @pl.when(nci < NCH)
                    def _():
                        _start(nci, 1 - b)
                    _wait(ci, b)
                    rows_v = rows_b[b]
                    a1_v = a1_b[b]
                    a2_v = a2_b[b]
                    for e in range(KC):
                        xx = a1_v[e, :] + a2_v[e, :]
                        exrow = jnp.exp(jnp.maximum(xx, 0.2 * xx))
                        if hp == 0:
                            ex_b[b][e, :] = exrow
                        cv = jnp.full((16,), exrow[hp])
                        for q in range(OC // 16):
                            rows_v[e, pl.ds(q * 16, 16)] = (
                                rows_v[e, pl.ds(q * 16, 16)] * cv)
                    dstv = idx_dl.at[ci, 0]
                    if hp == 0:
                        pltpu.async_copy(ex_b[b], denom_sh.at[dstv],
                                         sems[b], add=True)
                    pltpu.async_copy(rows_v, numer_sh.at[dstv],
                                     sems[b], add=True)
                return 0
            lax.fori_loop(0, NCH // 2, chunk2, 0)
            _scat_wait(1)
            plsc.subcore_barrier()

            def dblk(i, _):
                blk = sid + i * NS
                @pl.when(blk < nblk)
                def _():
                    pltpu.sync_copy(
                        numer_sh.at[pl.ds(blk * KC, KC)],
                        numer_out.at[c, r, hp, cid, pl.ds(blk * KC, KC)])
                    if hp == 0:
                        pltpu.sync_copy(
                            denom_sh.at[pl.ds(blk * KC, KC)],
                            denom_out.at[c, r, cid, pl.ds(blk * KC, KC)])
                return 0
            lax.fori_loop(0, (nblk + NS - 1) // NS, dblk, 0)
            plsc.subcore_barrier()
        return 0
    lax.fori_loop(0, C * R, cr_body, 0)


def _edge_kernel():
    mesh = plsc.VectorSubcoreMesh(core_axis_name="c", subcore_axis_name="s",
                                  num_cores=NC, num_subcores=NS)
    return pl.kernel(
        _edge_body,
        compiler_params=pltpu.CompilerParams(use_tc_tiling_on_sc=False),
        out_type=[
            jax.ShapeDtypeStruct((C, R, H, NC, NP, OC), jnp.float32),
            jax.ShapeDtypeStruct((C, R, NC, NP, 16), jnp.float32),
        ],
        mesh=mesh,
        scratch_types=[
            pltpu.VMEM((NCH, 1, KC), jnp.int32),
            pltpu.VMEM((NCH, 1, KC), jnp.int32),
            pltpu.VMEM((NCH, 1, KC), jnp.int32),
            pltpu.VMEM((NCH, 1, KC), jnp.int32),
            pltpu.VMEM((KC, OC), jnp.float32),
            pltpu.VMEM((KC, OC), jnp.float32),
            pltpu.VMEM((KC, 16), jnp.float32),
            pltpu.VMEM((KC, 16), jnp.float32),
            pltpu.VMEM((KC, 16), jnp.float32),
            pltpu.VMEM((KC, 16), jnp.float32),
            pltpu.VMEM((KC, 16), jnp.float32),
            pltpu.VMEM((KC, 16), jnp.float32),
            pltpu.VMEM((KC, 16), jnp.float32),
            pltpu.VMEM_SHARED((NP, OC), jnp.float32),
            pltpu.VMEM_SHARED((NP, 16), jnp.float32),
            pltpu.SemaphoreType.DMA,
            pltpu.SemaphoreType.DMA,
            pltpu.SemaphoreType.DMA,
            pltpu.SemaphoreType.DMA,
        ],
    )


def _edge_call(h_all, asrc, adst, srch, srcg, dstg, dstl):
    h3 = h_all.reshape(C * R * NP * H, OC)
    return _edge_kernel()(h3, asrc.reshape(C * R * NP, 16),
                          adst.reshape(C * R * NP, 16),
                          srch, srcg, dstg, dstl)


def _norm_body(num_ref, den_ref, h_ref, as_ref, ad_ref, bias_ref,
               sw_ref, sb_ref, sq_ref, z_ref, m_ref):
    b = pl.program_id(1)
    Os = []
    for r in range(R):
        dn = den_ref[0, r, 0] + den_ref[0, r, 1]
        asl = as_ref[0, r]
        adl = ad_ref[0, r]
        xx = asl + adl
        exs = jnp.exp(jnp.maximum(xx, 0.2 * xx))
        hrow = h_ref[0, r]
        cols = []
        for hh in range(H):
            sl = slice(hh * OC, (hh + 1) * OC)
            e_h = exs[:, hh:hh + 1]
            nm = num_ref[0, r, hh, 0] + num_ref[0, r, hh, 1]
            numer = nm + hrow[:, sl] * e_h
            denom = dn[:, hh:hh + 1] + e_h
            cols.append(numer / (denom + 1e-16))
        out = jnp.concatenate(cols, axis=1) + bias_ref[r:r + 1, :]
        Os.append(jax.nn.relu(out))
    sw = sw_ref[...]
    sb = sb_ref[0:1, :]
    sq = sq_ref[0:1, :]
    betas = []
    for r in range(R):
        w = jnp.tanh(jnp.dot(Os[r], sw, preferred_element_type=jnp.float32) + sb)
        betas.append(jnp.sum(w * sq, axis=1, keepdims=True))
    bmax = jnp.maximum(jnp.maximum(betas[0], betas[1]), betas[2])
    es = [jnp.exp(bb - bmax) for bb in betas]
    esum = es[0] + es[1] + es[2]
    z = (Os[0] * es[0] + Os[1] * es[1] + Os[2] * es[2]) / esum
    z_ref[0] = z
    rows = jax.lax.broadcasted_iota(jnp.int32, (BNN, 1), 0) + b * BNN
    zm = jnp.where(rows < N, z, 0.0)
    contrib = jnp.sum(zm, axis=0, keepdims=True) * (1.0 / N)
    @pl.when(b == 0)
    def _():
        m_ref[0] = contrib
    @pl.when(b != 0)
    def _():
        m_ref[0] = m_ref[0] + contrib


def _norm_call(numer, denom, h_all, asrc, adst, ppi_bias, sw, sb, sq):
    return pl.pallas_call(
        _norm_body,
        grid=(C, NBN),
        in_specs=[
            pl.BlockSpec((1, R, H, NC, BNN, OC), lambda c, b: (c, 0, 0, 0, b, 0)),
            pl.BlockSpec((1, R, NC, BNN, 16), lambda c, b: (c, 0, 0, b, 0)),
            pl.BlockSpec((1, R, BNN, HID), lambda c, b: (c, 0, b, 0)),
            pl.BlockSpec((1, R, BNN, 16), lambda c, b: (c, 0, b, 0)),
            pl.BlockSpec((1, R, BNN, 16), lambda c, b: (c, 0, b, 0)),
            pl.BlockSpec((R, HID), lambda c, b: (0, 0)),
            pl.BlockSpec((HID, SEM), lambda c, b: (0, 0)),
            pl.BlockSpec((1, SEM), lambda c, b: (0, 0)),
            pl.BlockSpec((1, SEM), lambda c, b: (0, 0)),
        ],
        out_specs=[
            pl.BlockSpec((1, BNN, HID), lambda c, b: (c, b, 0)),
            pl.BlockSpec((1, 1, HID), lambda c, b: (c, 0, 0)),
        ],
        out_shape=[
            jax.ShapeDtypeStruct((C, NP, HID), jnp.float32),
            jax.ShapeDtypeStruct((C, 1, HID), jnp.float32),
        ],
    )(numer, denom, h_all, asrc, adst, ppi_bias, sw, sb, sq)


def _meta_body(m_ref, nbr_ref, cnt_ref, mw_ref, mas_ref, mad_ref, mb_ref,
               sw_ref, sb_ref, sq_ref, out_ref):
    MP = 8
    meta0 = jnp.concatenate(
        [m_ref[...], jnp.zeros((MP - C, HID), jnp.float32)], axis=0)

    def smooth(_, meta):
        for ti in range(T):
            newrow = jnp.dot(nbr_ref[ti:ti + 1, :], meta,
                             preferred_element_type=jnp.float32)
            rmask = jax.lax.broadcasted_iota(jnp.int32, (MP, HID), 0) == (C + ti)
            meta = jnp.where(rmask, newrow, meta)
        return meta
    meta = lax.fori_loop(0, TISSUE_UPDATE, smooth, meta0)

    ones_col = jnp.ones((MP, 1), jnp.float32)
    Os = []
    for r in range(RM):
        hm = jnp.dot(meta, mw_ref[r], preferred_element_type=jnp.float32)
        tsr = hm * mas_ref[r:r + 1, :]
        tdr = hm * mad_ref[r:r + 1, :]
        cnt = cnt_ref[r]
        cols = []
        for hh in range(H):
            sl = slice(hh * OC, (hh + 1) * OC)
            asr = jnp.sum(tsr[:, sl], axis=1, keepdims=True)
            adr = jnp.sum(tdr[:, sl], axis=1, keepdims=True)
            asr_mat = lax.dot_general(ones_col, asr,
                                      (((1,), (1,)), ((), ())),
                                      preferred_element_type=jnp.float32)
            xx = asr_mat + adr
            wgt = cnt * jnp.exp(jnp.maximum(xx, 0.2 * xx))
            den = jnp.sum(wgt, axis=1, keepdims=True)
            num = jnp.dot(wgt, hm[:, sl], preferred_element_type=jnp.float32)
            cols.append(num / (den + 1e-16))
        out = jnp.concatenate(cols, axis=1) + mb_ref[r:r + 1, :]
        Os.append(jax.nn.relu(out))
    sw = sw_ref[...]
    sb = sb_ref[0:1, :]
    sq = sq_ref[0:1, :]
    betas = []
    for r in range(RM):
        w = jnp.tanh(jnp.dot(Os[r], sw, preferred_element_type=jnp.float32) + sb)
        betas.append(jnp.sum(w * sq, axis=1, keepdims=True))
    bmax = jnp.maximum(betas[0], betas[1])
    es = [jnp.exp(bb - bmax) for bb in betas]
    esum = es[0] + es[1]
    out_ref[...] = (Os[0] * es[0] + Os[1] * es[1]) / esum


def _meta_call(m, nbr_oh, cnt, meta_W, mas_flat, mad_flat, meta_bias, sw, sb, sq):
    return pl.pallas_call(
        _meta_body,
        out_shape=jax.ShapeDtypeStruct((8, HID), jnp.float32),
    )(m, nbr_oh, cnt, meta_W, mas_flat, mad_flat, meta_bias, sw, sb, sq)


def kernel(ppi_x, metagraph_x, ppi_edgetypes, metagraph_edgetypes,
           ppi_edge_index, metagraph_edge_index, tissue_neighbors,
           ppi_W, ppi_att_src, ppi_att_dst, ppi_bias,
           meta_W, meta_att_src, meta_att_dst, meta_bias,
           sem_W, sem_b, sem_q, init_cci):
    f32 = jnp.float32
    ppi_x = ppi_x.astype(f32)
    xp = jnp.pad(ppi_x, ((0, 0), (0, NP - N), (0, 0)))

    et = ppi_edgetypes.astype(jnp.int32)
    pad_e = NW * EPT - E
    src = jnp.pad(et[:, :, 0, :], ((0, 0), (0, 0), (0, pad_e)),
                  constant_values=TRASH)
    dst = jnp.pad(et[:, :, 1, :], ((0, 0), (0, 0), (0, pad_e)),
                  constant_values=TRASH)
    offs = (jnp.arange(C)[:, None] * R + jnp.arange(R)[None, :]) * NP
    gsrc = src + offs[:, :, None]
    gdst = dst + offs[:, :, None]
    srcg = gsrc.reshape(C, R, NW, NCH, 1, KC)
    dstg = gdst.reshape(C, R, NW, NCH, 1, KC)
    dstl = dst.reshape(C, R, NW, NCH, 1, KC)
    srch = (gsrc[:, :, None, :] * H
            + jnp.arange(H)[None, None, :, None]).reshape(
                C, R, H, NW, NCH, 1, KC)

    att_src_flat = ppi_att_src.reshape(R, HID).astype(f32)
    att_dst_flat = ppi_att_dst.reshape(R, HID).astype(f32)
    sw = sem_W.reshape(HID, SEM).astype(f32)
    sb = sem_b.reshape(1, SEM).astype(f32)
    sq = sem_q.reshape(1, SEM).astype(f32)

    h_all, asrc, adst = _prep_call(xp, ppi_W.astype(f32),
                                   att_src_flat, att_dst_flat)
    numer, denom = _edge_call(h_all, asrc, adst, srch, srcg, dstg, dstl)
    z, m = _norm_call(numer, denom, h_all, asrc, adst,
                      ppi_bias.astype(f32), sw, sb, sq)
    m = m.reshape(C, HID)

    rows8 = jnp.arange(8)
    nbr_oh = (jnp.sum((rows8[None, None, :] == tissue_neighbors[:, :, None])
                      .astype(f32), axis=1) / 3.0)
    me = metagraph_edgetypes.astype(jnp.int32)
    cnt = jnp.sum((rows8[None, :, None, None] == me[:, None, None, 1, :])
                  & (rows8[None, None, :, None] == me[:, None, None, 0, :]),
                  axis=-1).astype(f32)
    cnt = cnt + jnp.eye(8, dtype=f32)[None]

    meta_out = _meta_call(m, nbr_oh, cnt, meta_W.astype(f32),
                          meta_att_src.reshape(RM, HID).astype(f32),
                          meta_att_dst.reshape(RM, HID).astype(f32),
                          meta_bias.astype(f32), sw, sb, sq)

    return z[:, :N, :], meta_out[:M]


# speedup vs baseline: 41.6526x; 1.0001x over previous
import functools

import jax
import jax.numpy as jnp
from jax import lax
from jax.experimental import pallas as pl
from jax.experimental.pallas import tpu as pltpu
from jax.experimental.pallas import tpu_sc as plsc

C = 4; N = 10000; D = 128; R = 3; E = 160000
H = 3; OC = 64; HID = H * OC
RM = 2; T = 3; M = C + T; EM = 20; SEM = 128
TISSUE_UPDATE = 100

NP = 10048
TRASH = 10008
NC = 2; NS = 16; NW = NC * NS
RPT = NP // NW
KC = 64
EPT = 5120
NCH = EPT // KC
BN = 2512
NB = NP // BN
BNN = 1256
NBN = NP // BNN


def _prep_body(x_ref, w_ref, asf_ref, adf_ref, h_ref, as_ref, ad_ref):
    x = x_ref[0]
    w = w_ref[0]
    h = jnp.dot(x, w, preferred_element_type=jnp.float32)
    h_ref[0, 0] = h
    r = pl.program_id(1)
    asf = asf_ref[pl.ds(r, 1), :]
    adf = adf_ref[pl.ds(r, 1), :]
    ts = h * asf
    td = h * adf
    cols_s = []
    cols_d = []
    for hh in range(H):
        sl = slice(hh * OC, (hh + 1) * OC)
        cols_s.append(jnp.sum(ts[:, sl], axis=1, keepdims=True))
        cols_d.append(jnp.sum(td[:, sl], axis=1, keepdims=True))
    z = jnp.zeros((x.shape[0], 16 - H), jnp.float32)
    as_ref[0, 0] = jnp.concatenate(cols_s + [z], axis=1)
    ad_ref[0, 0] = jnp.concatenate(cols_d + [z], axis=1)


def _prep_call(xp, ppi_W, att_src_flat, att_dst_flat):
    return pl.pallas_call(
        _prep_body,
        grid=(C, R, NB),
        in_specs=[
            pl.BlockSpec((1, BN, D), lambda c, r, b: (c, b, 0)),
            pl.BlockSpec((1, D, HID), lambda c, r, b: (r, 0, 0)),
            pl.BlockSpec((R, HID), lambda c, r, b: (0, 0)),
            pl.BlockSpec((R, HID), lambda c, r, b: (0, 0)),
        ],
        out_specs=[
            pl.BlockSpec((1, 1, BN, HID), lambda c, r, b: (c, r, b, 0)),
            pl.BlockSpec((1, 1, BN, 16), lambda c, r, b: (c, r, b, 0)),
            pl.BlockSpec((1, 1, BN, 16), lambda c, r, b: (c, r, b, 0)),
        ],
        out_shape=[
            jax.ShapeDtypeStruct((C, R, NP, HID), jnp.float32),
            jax.ShapeDtypeStruct((C, R, NP, 16), jnp.float32),
            jax.ShapeDtypeStruct((C, R, NP, 16), jnp.float32),
        ],
    )(xp, ppi_W, att_src_flat, att_dst_flat)


def _edge_body(h3_hbm, asrc_hbm, adst_hbm, srch_hbm, srcg_hbm, dstg_hbm,
               dstl_hbm, numer_out, denom_out,
               idx_h, idx_g, idx_dg, idx_dl, rows_v0, rows_v1, a1_v0, a1_v1,
               a2_v0, a2_v1, ex_v0, ex_v1, zden_v, numer_sh, denom_sh,
               sem0, sem1, sem2, sem3):
    rows_b = [rows_v0, rows_v1]
    a1_b = [a1_v0, a1_v1]
    a2_b = [a2_v0, a2_v1]
    ex_b = [ex_v0, ex_v1]
    sem_b = [sem0, sem1]
    sems = [sem2, sem3]
    cid = lax.axis_index("c")
    sid = lax.axis_index("s")
    tile = cid * NS + sid
    nblk = NP // KC
    f32 = jnp.float32

    def _zden(i, _):
        zden_v[i, :] = jnp.zeros((16,), f32)
        return 0
    lax.fori_loop(0, KC, _zden, 0)

    def cr_body(j, _):
        c = j // R
        r = j - c * R
        pltpu.sync_copy(srcg_hbm.at[c, r, tile], idx_g)
        pltpu.sync_copy(dstg_hbm.at[c, r, tile], idx_dg)
        pltpu.sync_copy(dstl_hbm.at[c, r, tile], idx_dl)

        for hp in range(H):
            def _zrows(i, _):
                rr = i // (OC // 16)
                qq = i - rr * (OC // 16)
                rows_v0[rr, pl.ds(qq * 16, 16)] = jnp.zeros((16,), f32)
                return 0
            lax.fori_loop(0, KC * (OC // 16), _zrows, 0)

            def zblk(i, _):
                blk = sid + i * NS
                @pl.when(blk < nblk)
                def _():
                    pltpu.sync_copy(rows_v0, numer_sh.at[pl.ds(blk * KC, KC)])
                    if hp == 0:
                        pltpu.sync_copy(zden_v,
                                        denom_sh.at[pl.ds(blk * KC, KC)])
                return 0
            lax.fori_loop(0, (nblk + NS - 1) // NS, zblk, 0)
            pltpu.sync_copy(srch_hbm.at[c, r, hp, tile], idx_h)
            plsc.subcore_barrier()

            def _start(ci, b):
                pltpu.async_copy(h3_hbm.at[idx_h.at[ci, 0]], rows_b[b],
                                 sem_b[b])
                pltpu.async_copy(asrc_hbm.at[idx_g.at[ci, 0]], a1_b[b],
                                 sem_b[b])
                pltpu.async_copy(adst_hbm.at[idx_dg.at[ci, 0]], a2_b[b],
                                 sem_b[b])

            def _wait(ci, b):
                pltpu.make_async_copy(h3_hbm.at[idx_h.at[ci, 0]], rows_b[b],
                                      sem_b[b]).wait()
                pltpu.make_async_copy(asrc_hbm.at[idx_g.at[ci, 0]],
                                      a1_b[b], sem_b[b]).wait()
                pltpu.make_async_copy(adst_hbm.at[idx_dg.at[ci, 0]],
                                      a2_b[b], sem_b[b]).wait()

            def _scat_wait(x):
                pltpu.make_async_copy(rows_b[x],
                                      numer_sh.at[idx_dl.at[0, 0]],
                                      sems[x]).wait()
                if hp == 0:
                    pltpu.make_async_copy(ex_b[x],
                                          denom_sh.at[idx_dl.at[0, 0]],
                                          sems[x]).wait()

            _start(0, 0)
            def chunk2(cio, _):
                for b in range(2):
                    ci = cio * 2 + b
                    nci = ci + 1
                    @pl.when(ci > 0)
                    def _():
                        _scat_wait(1 - b)
                    @pl.when(nci < NCH)
                    def _():
                        _start(nci, 1 - b)
                    _wait(ci, b)
                    rows_v = rows_b[b]
                    a1_v = a1_b[b]
                    a2_v = a2_b[b]
                    hv = jnp.full((16, 1), hp, jnp.int32)
                    dnums = lax.GatherDimensionNumbers(
                        offset_dims=(), collapsed_slice_dims=(0,),
                        start_index_map=(0,))
                    for e in range(KC):
                        xx = a1_v[e, :] + a2_v[e, :]
                        exrow = jnp.exp(jnp.maximum(xx, 0.2 * xx))
                        if hp == 0:
                            ex_b[b][e, :] = exrow
                        cv = lax.gather(
                            exrow, hv, dnums, (1,),
                            mode=lax.GatherScatterMode.PROMISE_IN_BOUNDS)
                        for q in range(OC // 16):
                            rows_v[e, pl.ds(q * 16, 16)] = (
                                rows_v[e, pl.ds(q * 16, 16)] * cv)
                    dstv = idx_dl.at[ci, 0]
                    if hp == 0:
                        pltpu.async_copy(ex_b[b], denom_sh.at[dstv],
                                         sems[b], add=True)
                    pltpu.async_copy(rows_v, numer_sh.at[dstv],
                                     sems[b], add=True)
                return 0
            lax.fori_loop(0, NCH // 2, chunk2, 0)
            _scat_wait(1)
            plsc.subcore_barrier()

            def dblk(i, _):
                blk = sid + i * NS
                @pl.when(blk < nblk)
                def _():
                    pltpu.sync_copy(
                        numer_sh.at[pl.ds(blk * KC, KC)],
                        numer_out.at[c, r, hp, cid, pl.ds(blk * KC, KC)])
                    if hp == 0:
                        pltpu.sync_copy(
                            denom_sh.at[pl.ds(blk * KC, KC)],
                            denom_out.at[c, r, cid, pl.ds(blk * KC, KC)])
                return 0
            lax.fori_loop(0, (nblk + NS - 1) // NS, dblk, 0)
            plsc.subcore_barrier()
        return 0
    lax.fori_loop(0, C * R, cr_body, 0)


def _edge_kernel():
    mesh = plsc.VectorSubcoreMesh(core_axis_name="c", subcore_axis_name="s",
                                  num_cores=NC, num_subcores=NS)
    return pl.kernel(
        _edge_body,
        compiler_params=pltpu.CompilerParams(use_tc_tiling_on_sc=False),
        out_type=[
            jax.ShapeDtypeStruct((C, R, H, NC, NP, OC), jnp.float32),
            jax.ShapeDtypeStruct((C, R, NC, NP, 16), jnp.float32),
        ],
        mesh=mesh,
        scratch_types=[
            pltpu.VMEM((NCH, 1, KC), jnp.int32),
            pltpu.VMEM((NCH, 1, KC), jnp.int32),
            pltpu.VMEM((NCH, 1, KC), jnp.int32),
            pltpu.VMEM((NCH, 1, KC), jnp.int32),
            pltpu.VMEM((KC, OC), jnp.float32),
            pltpu.VMEM((KC, OC), jnp.float32),
            pltpu.VMEM((KC, 16), jnp.float32),
            pltpu.VMEM((KC, 16), jnp.float32),
            pltpu.VMEM((KC, 16), jnp.float32),
            pltpu.VMEM((KC, 16), jnp.float32),
            pltpu.VMEM((KC, 16), jnp.float32),
            pltpu.VMEM((KC, 16), jnp.float32),
            pltpu.VMEM((KC, 16), jnp.float32),
            pltpu.VMEM_SHARED((NP, OC), jnp.float32),
            pltpu.VMEM_SHARED((NP, 16), jnp.float32),
            pltpu.SemaphoreType.DMA,
            pltpu.SemaphoreType.DMA,
            pltpu.SemaphoreType.DMA,
            pltpu.SemaphoreType.DMA,
        ],
    )


def _edge_call(h_all, asrc, adst, srch, srcg, dstg, dstl):
    h3 = h_all.reshape(C * R * NP * H, OC)
    return _edge_kernel()(h3, asrc.reshape(C * R * NP, 16),
                          adst.reshape(C * R * NP, 16),
                          srch, srcg, dstg, dstl)


def _norm_body(num_ref, den_ref, h_ref, as_ref, ad_ref, bias_ref,
               sw_ref, sb_ref, sq_ref, z_ref, m_ref):
    b = pl.program_id(1)
    Os = []
    for r in range(R):
        dn = den_ref[0, r, 0] + den_ref[0, r, 1]
        asl = as_ref[0, r]
        adl = ad_ref[0, r]
        xx = asl + adl
        exs = jnp.exp(jnp.maximum(xx, 0.2 * xx))
        hrow = h_ref[0, r]
        cols = []
        for hh in range(H):
            sl = slice(hh * OC, (hh + 1) * OC)
            e_h = exs[:, hh:hh + 1]
            nm = num_ref[0, r, hh, 0] + num_ref[0, r, hh, 1]
            numer = nm + hrow[:, sl] * e_h
            denom = dn[:, hh:hh + 1] + e_h
            cols.append(numer / (denom + 1e-16))
        out = jnp.concatenate(cols, axis=1) + bias_ref[r:r + 1, :]
        Os.append(jax.nn.relu(out))
    sw = sw_ref[...]
    sb = sb_ref[0:1, :]
    sq = sq_ref[0:1, :]
    betas = []
    for r in range(R):
        w = jnp.tanh(jnp.dot(Os[r], sw, preferred_element_type=jnp.float32) + sb)
        betas.append(jnp.sum(w * sq, axis=1, keepdims=True))
    bmax = jnp.maximum(jnp.maximum(betas[0], betas[1]), betas[2])
    es = [jnp.exp(bb - bmax) for bb in betas]
    esum = es[0] + es[1] + es[2]
    z = (Os[0] * es[0] + Os[1] * es[1] + Os[2] * es[2]) / esum
    z_ref[0] = z
    rows = jax.lax.broadcasted_iota(jnp.int32, (BNN, 1), 0) + b * BNN
    zm = jnp.where(rows < N, z, 0.0)
    contrib = jnp.sum(zm, axis=0, keepdims=True) * (1.0 / N)
    @pl.when(b == 0)
    def _():
        m_ref[0] = contrib
    @pl.when(b != 0)
    def _():
        m_ref[0] = m_ref[0] + contrib


def _norm_call(numer, denom, h_all, asrc, adst, ppi_bias, sw, sb, sq):
    return pl.pallas_call(
        _norm_body,
        grid=(C, NBN),
        in_specs=[
            pl.BlockSpec((1, R, H, NC, BNN, OC), lambda c, b: (c, 0, 0, 0, b, 0)),
            pl.BlockSpec((1, R, NC, BNN, 16), lambda c, b: (c, 0, 0, b, 0)),
            pl.BlockSpec((1, R, BNN, HID), lambda c, b: (c, 0, b, 0)),
            pl.BlockSpec((1, R, BNN, 16), lambda c, b: (c, 0, b, 0)),
            pl.BlockSpec((1, R, BNN, 16), lambda c, b: (c, 0, b, 0)),
            pl.BlockSpec((R, HID), lambda c, b: (0, 0)),
            pl.BlockSpec((HID, SEM), lambda c, b: (0, 0)),
            pl.BlockSpec((1, SEM), lambda c, b: (0, 0)),
            pl.BlockSpec((1, SEM), lambda c, b: (0, 0)),
        ],
        out_specs=[
            pl.BlockSpec((1, BNN, HID), lambda c, b: (c, b, 0)),
            pl.BlockSpec((1, 1, HID), lambda c, b: (c, 0, 0)),
        ],
        out_shape=[
            jax.ShapeDtypeStruct((C, NP, HID), jnp.float32),
            jax.ShapeDtypeStruct((C, 1, HID), jnp.float32),
        ],
    )(numer, denom, h_all, asrc, adst, ppi_bias, sw, sb, sq)


def _meta_body(m_ref, nbr_ref, cnt_ref, mw_ref, mas_ref, mad_ref, mb_ref,
               sw_ref, sb_ref, sq_ref, out_ref):
    MP = 8
    meta0 = jnp.concatenate(
        [m_ref[...], jnp.zeros((MP - C, HID), jnp.float32)], axis=0)

    def smooth(_, meta):
        for ti in range(T):
            newrow = jnp.dot(nbr_ref[ti:ti + 1, :], meta,
                             preferred_element_type=jnp.float32)
            rmask = jax.lax.broadcasted_iota(jnp.int32, (MP, HID), 0) == (C + ti)
            meta = jnp.where(rmask, newrow, meta)
        return meta
    meta = lax.fori_loop(0, TISSUE_UPDATE, smooth, meta0)

    ones_col = jnp.ones((MP, 1), jnp.float32)
    Os = []
    for r in range(RM):
        hm = jnp.dot(meta, mw_ref[r], preferred_element_type=jnp.float32)
        tsr = hm * mas_ref[r:r + 1, :]
        tdr = hm * mad_ref[r:r + 1, :]
        cnt = cnt_ref[r]
        cols = []
        for hh in range(H):
            sl = slice(hh * OC, (hh + 1) * OC)
            asr = jnp.sum(tsr[:, sl], axis=1, keepdims=True)
            adr = jnp.sum(tdr[:, sl], axis=1, keepdims=True)
            asr_mat = lax.dot_general(ones_col, asr,
                                      (((1,), (1,)), ((), ())),
                                      preferred_element_type=jnp.float32)
            xx = asr_mat + adr
            wgt = cnt * jnp.exp(jnp.maximum(xx, 0.2 * xx))
            den = jnp.sum(wgt, axis=1, keepdims=True)
            num = jnp.dot(wgt, hm[:, sl], preferred_element_type=jnp.float32)
            cols.append(num / (den + 1e-16))
        out = jnp.concatenate(cols, axis=1) + mb_ref[r:r + 1, :]
        Os.append(jax.nn.relu(out))
    sw = sw_ref[...]
    sb = sb_ref[0:1, :]
    sq = sq_ref[0:1, :]
    betas = []
    for r in range(RM):
        w = jnp.tanh(jnp.dot(Os[r], sw, preferred_element_type=jnp.float32) + sb)
        betas.append(jnp.sum(w * sq, axis=1, keepdims=True))
    bmax = jnp.maximum(betas[0], betas[1])
    es = [jnp.exp(bb - bmax) for bb in betas]
    esum = es[0] + es[1]
    out_ref[...] = (Os[0] * es[0] + Os[1] * es[1]) / esum


def _meta_call(m, nbr_oh, cnt, meta_W, mas_flat, mad_flat, meta_bias, sw, sb, sq):
    return pl.pallas_call(
        _meta_body,
        out_shape=jax.ShapeDtypeStruct((8, HID), jnp.float32),
    )(m, nbr_oh, cnt, meta_W, mas_flat, mad_flat, meta_bias, sw, sb, sq)


def kernel(ppi_x, metagraph_x, ppi_edgetypes, metagraph_edgetypes,
           ppi_edge_index, metagraph_edge_index, tissue_neighbors,
           ppi_W, ppi_att_src, ppi_att_dst, ppi_bias,
           meta_W, meta_att_src, meta_att_dst, meta_bias,
           sem_W, sem_b, sem_q, init_cci):
    f32 = jnp.float32
    ppi_x = ppi_x.astype(f32)
    xp = jnp.pad(ppi_x, ((0, 0), (0, NP - N), (0, 0)))

    et = ppi_edgetypes.astype(jnp.int32)
    pad_e = NW * EPT - E
    src = jnp.pad(et[:, :, 0, :], ((0, 0), (0, 0), (0, pad_e)),
                  constant_values=TRASH)
    dst = jnp.pad(et[:, :, 1, :], ((0, 0), (0, 0), (0, pad_e)),
                  constant_values=TRASH)
    offs = (jnp.arange(C)[:, None] * R + jnp.arange(R)[None, :]) * NP
    gsrc = src + offs[:, :, None]
    gdst = dst + offs[:, :, None]
    srcg = gsrc.reshape(C, R, NW, NCH, 1, KC)
    dstg = gdst.reshape(C, R, NW, NCH, 1, KC)
    dstl = dst.reshape(C, R, NW, NCH, 1, KC)
    srch = (gsrc[:, :, None, :] * H
            + jnp.arange(H)[None, None, :, None]).reshape(
                C, R, H, NW, NCH, 1, KC)

    att_src_flat = ppi_att_src.reshape(R, HID).astype(f32)
    att_dst_flat = ppi_att_dst.reshape(R, HID).astype(f32)
    sw = sem_W.reshape(HID, SEM).astype(f32)
    sb = sem_b.reshape(1, SEM).astype(f32)
    sq = sem_q.reshape(1, SEM).astype(f32)

    h_all, asrc, adst = _prep_call(xp, ppi_W.astype(f32),
                                   att_src_flat, att_dst_flat)
    numer, denom = _edge_call(h_all, asrc, adst, srch, srcg, dstg, dstl)
    z, m = _norm_call(numer, denom, h_all, asrc, adst,
                      ppi_bias.astype(f32), sw, sb, sq)
    m = m.reshape(C, HID)

    rows8 = jnp.arange(8)
    nbr_oh = (jnp.sum((rows8[None, None, :] == tissue_neighbors[:, :, None])
                      .astype(f32), axis=1) / 3.0)
    me = metagraph_edgetypes.astype(jnp.int32)
    cnt = jnp.sum((rows8[None, :, None, None] == me[:, None, None, 1, :])
                  & (rows8[None, None, :, None] == me[:, None, None, 0, :]),
                  axis=-1).astype(f32)
    cnt = cnt + jnp.eye(8, dtype=f32)[None]

    meta_out = _meta_call(m, nbr_oh, cnt, meta_W.astype(f32),
                          meta_att_src.reshape(RM, HID).astype(f32),
                          meta_att_dst.reshape(RM, HID).astype(f32),
                          meta_bias.astype(f32), sw, sb, sq)

    return z[:, :N, :], meta_out[:M]
